# baseline (device time: 122306 ns/iter reference)
import jax
import jax.numpy as jnp
from jax import lax
from jax.experimental import pallas as pl
from jax.experimental.pallas import tpu as pltpu

N_DEV = 8


def kernel(x, W1, W2):
    m, k = x.shape
    _, d = W1.shape
    _, f = W2.shape
    chunk = m // N_DEV

    def body(x_ref, w1_ref, w2_ref, out_ref,
             rs_buf, ag_buf, rs_send, rs_recv, ag_send, ag_recv):
        i = lax.axis_index("i")
        left = lax.rem(i - 1 + N_DEV, N_DEV)
        right = lax.rem(i + 1, N_DEV)

        barrier_sem = pltpu.get_barrier_semaphore()
        for nbr in (left, right):
            pl.semaphore_signal(
                barrier_sem, inc=1,
                device_id=(nbr,), device_id_type=pl.DeviceIdType.MESH,
            )
        pl.semaphore_wait(barrier_sem, 2)

        w1 = w1_ref[...].astype(jnp.bfloat16)
        w2 = w2_ref[...].astype(jnp.bfloat16)

        def partial_chunk(c):
            xa = x_ref[pl.ds(c * chunk, chunk), :].astype(jnp.bfloat16)
            return jnp.dot(xa, w1, preferred_element_type=jnp.float32)

        c0 = lax.rem(i - 1 + N_DEV, N_DEV)
        rs_buf[N_DEV - 1] = partial_chunk(c0).astype(jnp.bfloat16)

        acc = None
        for s in range(N_DEV - 1):
            src_slot = (N_DEV - 1) if s == 0 else (s - 1)
            rdma = pltpu.make_async_remote_copy(
                src_ref=rs_buf.at[src_slot],
                dst_ref=rs_buf.at[s],
                send_sem=rs_send.at[s],
                recv_sem=rs_recv.at[s],
                device_id=(right,),
                device_id_type=pl.DeviceIdType.MESH,
            )
            rdma.start()
            rdma.wait()
            c = lax.rem(i - 2 - s + 2 * N_DEV, N_DEV)
            acc = rs_buf[s].astype(jnp.float32) + partial_chunk(c)
            rs_buf[s] = acc.astype(jnp.bfloat16)

        out_ref[pl.ds(i * chunk, chunk), :] = jnp.dot(
            acc.astype(jnp.bfloat16), w2, preferred_element_type=jnp.float32
        )

        for s in range(N_DEV - 1):
            src = rs_buf.at[N_DEV - 2] if s == 0 else ag_buf.at[s - 1]
            rdma = pltpu.make_async_remote_copy(
                src_ref=src,
                dst_ref=ag_buf.at[s],
                send_sem=ag_send.at[s],
                recv_sem=ag_recv.at[s],
                device_id=(right,),
                device_id_type=pl.DeviceIdType.MESH,
            )
            rdma.start()
            rdma.wait()
            cr = lax.rem(i - 1 - s + 2 * N_DEV, N_DEV)
            out_ref[pl.ds(cr * chunk, chunk), :] = jnp.dot(
                ag_buf[s], w2, preferred_element_type=jnp.float32
            )

    return pl.pallas_call(
        body,
        out_shape=jax.ShapeDtypeStruct((m, f), jnp.float32),
        in_specs=[
            pl.BlockSpec(memory_space=pltpu.VMEM),
            pl.BlockSpec(memory_space=pltpu.VMEM),
            pl.BlockSpec(memory_space=pltpu.VMEM),
        ],
        out_specs=pl.BlockSpec(memory_space=pltpu.VMEM),
        scratch_shapes=[
            pltpu.VMEM((N_DEV, chunk, d), jnp.bfloat16),
            pltpu.VMEM((N_DEV - 1, chunk, d), jnp.bfloat16),
            pltpu.SemaphoreType.DMA((N_DEV - 1,)),
            pltpu.SemaphoreType.DMA((N_DEV - 1,)),
            pltpu.SemaphoreType.DMA((N_DEV - 1,)),
            pltpu.SemaphoreType.DMA((N_DEV - 1,)),
        ],
        compiler_params=pltpu.CompilerParams(collective_id=0),
    )(x, W1, W2)


# device time: 78589 ns/iter; 1.5563x vs baseline; 1.5563x over previous
import jax
import jax.numpy as jnp
from jax import lax
from jax.experimental import pallas as pl
from jax.experimental.pallas import tpu as pltpu

N_DEV = 8


def kernel(x, W1, W2):
    m, k = x.shape
    _, d = W1.shape
    _, f = W2.shape
    chunk = m // N_DEV
    hd = d // 2

    def body(x_ref, w1_ref, w2_ref, out_ref, part_ref,
             rsA, rsB, agA, agB,
             rsA_s, rsA_r, rsB_s, rsB_r, agA_s, agA_r, agB_s, agB_r):
        i = lax.axis_index("i")
        left = lax.rem(i - 1 + N_DEV, N_DEV)
        right = lax.rem(i + 1, N_DEV)

        barrier_sem = pltpu.get_barrier_semaphore()
        for nbr in (left, right):
            pl.semaphore_signal(
                barrier_sem, inc=1,
                device_id=(nbr,), device_id_type=pl.DeviceIdType.MESH,
            )
        pl.semaphore_wait(barrier_sem, 2)

        w1 = w1_ref[...].astype(jnp.bfloat16)
        w2A = w2_ref[:hd, :].astype(jnp.bfloat16)
        w2B = w2_ref[hd:, :].astype(jnp.bfloat16)

        def pchunk(c):
            xa = x_ref[pl.ds(c * chunk, chunk), :].astype(jnp.bfloat16)
            return jnp.dot(xa, w1, preferred_element_type=jnp.float32)

        def rdma(src, dst, ssem, rsem, dev):
            return pltpu.make_async_remote_copy(
                src_ref=src, dst_ref=dst, send_sem=ssem, recv_sem=rsem,
                device_id=(dev,), device_id_type=pl.DeviceIdType.MESH,
            )

        sent = []

        p7 = pchunk(lax.rem(i + 7, N_DEV))
        part_ref[7] = p7
        rsA[7] = p7[:, :hd].astype(jnp.bfloat16)
        ra = rdma(rsA.at[7], rsA.at[0], rsA_s.at[0], rsA_r.at[0], right)
        ra.start()
        sent.append(ra)
        p1 = pchunk(lax.rem(i + 1, N_DEV))
        part_ref[1] = p1
        rsB[7] = p1[:, hd:].astype(jnp.bfloat16)
        rb = rdma(rsB.at[7], rsB.at[0], rsB_s.at[0], rsB_r.at[0], left)
        rb.start()
        sent.append(rb)

        for r_off in (6, 2, 5, 3, 4, 0):
            part_ref[r_off] = pchunk(lax.rem(i + r_off, N_DEV))

        accA = accB = None
        for s in range(N_DEV - 1):
            rdma(rsA.at[s], rsA.at[s], rsA_s.at[s], rsA_r.at[s],
                 right).wait_recv()
            accA = rsA[s].astype(jnp.float32) + part_ref[6 - s][:, :hd]
            if s < N_DEV - 2:
                rsA[s] = accA.astype(jnp.bfloat16)
                ra = rdma(rsA.at[s], rsA.at[s + 1],
                          rsA_s.at[s + 1], rsA_r.at[s + 1], right)
                ra.start()
                sent.append(ra)
            rdma(rsB.at[s], rsB.at[s], rsB_s.at[s], rsB_r.at[s],
                 left).wait_recv()
            accB = rsB[s].astype(jnp.float32) + part_ref[(2 + s) % N_DEV][:, hd:]
            if s < N_DEV - 2:
                rsB[s] = accB.astype(jnp.bfloat16)
                rb = rdma(rsB.at[s], rsB.at[s + 1],
                          rsB_s.at[s + 1], rsB_r.at[s + 1], left)
                rb.start()
                sent.append(rb)

        agA[7] = accA.astype(jnp.bfloat16)
        ra = rdma(agA.at[7], agA.at[0], agA_s.at[0], agA_r.at[0], right)
        ra.start()
        sent.append(ra)
        agB[7] = accB.astype(jnp.bfloat16)
        rb = rdma(agB.at[7], agB.at[0], agB_s.at[0], agB_r.at[0], left)
        rb.start()
        sent.append(rb)
        out_ref[pl.ds(i * chunk, chunk), :] = (
            jnp.dot(agA[7], w2A, preferred_element_type=jnp.float32)
            + jnp.dot(agB[7], w2B, preferred_element_type=jnp.float32)
        )

        for s in range(N_DEV - 1):
            rdma(agA.at[s], agA.at[s], agA_s.at[s], agA_r.at[s],
                 right).wait_recv()
            if s < N_DEV - 2:
                ra = rdma(agA.at[s], agA.at[s + 1],
                          agA_s.at[s + 1], agA_r.at[s + 1], right)
                ra.start()
                sent.append(ra)
            rdma(agB.at[s], agB.at[s], agB_s.at[s], agB_r.at[s],
                 left).wait_recv()
            if s < N_DEV - 2:
                rb = rdma(agB.at[s], agB.at[s + 1],
                          agB_s.at[s + 1], agB_r.at[s + 1], left)
                rb.start()
                sent.append(rb)
            pieceA = jnp.dot(agA[s], w2A, preferred_element_type=jnp.float32)
            pieceB = jnp.dot(agB[s], w2B, preferred_element_type=jnp.float32)
            cA = lax.rem(i - 1 - s + 2 * N_DEV, N_DEV)
            cB = lax.rem(i + 1 + s, N_DEV)
            dsA = pl.ds(cA * chunk, chunk)
            dsB = pl.ds(cB * chunk, chunk)
            if s < 3:
                out_ref[dsA, :] = pieceA
                out_ref[dsB, :] = pieceB
            elif s == 3:
                out_ref[dsA, :] = pieceA + pieceB
            else:
                out_ref[dsA, :] = out_ref[dsA, :] + pieceA
                out_ref[dsB, :] = out_ref[dsB, :] + pieceB

        for r in sent:
            r.wait_send()

    return pl.pallas_call(
        body,
        out_shape=jax.ShapeDtypeStruct((m, f), jnp.float32),
        in_specs=[
            pl.BlockSpec(memory_space=pltpu.VMEM),
            pl.BlockSpec(memory_space=pltpu.VMEM),
            pl.BlockSpec(memory_space=pltpu.VMEM),
        ],
        out_specs=pl.BlockSpec(memory_space=pltpu.VMEM),
        scratch_shapes=[
            pltpu.VMEM((N_DEV, chunk, d), jnp.float32),
            pltpu.VMEM((N_DEV, chunk, hd), jnp.bfloat16),
            pltpu.VMEM((N_DEV, chunk, hd), jnp.bfloat16),
            pltpu.VMEM((N_DEV, chunk, hd), jnp.bfloat16),
            pltpu.VMEM((N_DEV, chunk, hd), jnp.bfloat16),
            pltpu.SemaphoreType.DMA((N_DEV - 1,)),
            pltpu.SemaphoreType.DMA((N_DEV - 1,)),
            pltpu.SemaphoreType.DMA((N_DEV - 1,)),
            pltpu.SemaphoreType.DMA((N_DEV - 1,)),
            pltpu.SemaphoreType.DMA((N_DEV - 1,)),
            pltpu.SemaphoreType.DMA((N_DEV - 1,)),
            pltpu.SemaphoreType.DMA((N_DEV - 1,)),
            pltpu.SemaphoreType.DMA((N_DEV - 1,)),
        ],
        compiler_params=pltpu.CompilerParams(collective_id=0),
    )(x, W1, W2)


# device time: 62119 ns/iter; 1.9689x vs baseline; 1.2651x over previous
import jax
import jax.numpy as jnp
from jax import lax
from jax.experimental import pallas as pl
from jax.experimental.pallas import tpu as pltpu

N_DEV = 8
NSUB = 2
LANES = 2 * NSUB


def kernel(x, W1, W2):
    m, k = x.shape
    _, d = W1.shape
    _, f = W2.shape
    chunk = m // N_DEV
    lw = d // LANES

    lane_order = []
    for s in range(NSUB):
        lane_order += [s, NSUB + s]

    def body(x_ref, w1_ref, w2_ref, out_ref, part_ref,
             rs_buf, ag_buf, rs_s, rs_r, ag_s, ag_r):
        i = lax.axis_index("i")
        left = lax.rem(i - 1 + N_DEV, N_DEV)
        right = lax.rem(i + 1, N_DEV)

        barrier_sem = pltpu.get_barrier_semaphore()
        for nbr in (left, right):
            pl.semaphore_signal(
                barrier_sem, inc=1,
                device_id=(nbr,), device_id_type=pl.DeviceIdType.MESH,
            )
        pl.semaphore_wait(barrier_sem, 2)

        w1 = w1_ref[...].astype(jnp.bfloat16)
        w2l = [w2_ref[l * lw:(l + 1) * lw, :].astype(jnp.bfloat16)
               for l in range(LANES)]

        def rightward(l):
            return l < NSUB

        def dev(l):
            return right if rightward(l) else left

        def pchunk(c):
            xa = x_ref[pl.ds(c * chunk, chunk), :].astype(jnp.bfloat16)
            return jnp.dot(xa, w1, preferred_element_type=jnp.float32)

        def rdma(buf, sems_s, sems_r, l, src_slot, dst_slot, step):
            return pltpu.make_async_remote_copy(
                src_ref=buf.at[l, src_slot],
                dst_ref=buf.at[l, dst_slot],
                send_sem=sems_s.at[l, step],
                recv_sem=sems_r.at[l, step],
                device_id=(dev(l),), device_id_type=pl.DeviceIdType.MESH,
            )

        sent = []

        p7 = pchunk(lax.rem(i + 7, N_DEV))
        part_ref[7] = p7
        p1 = pchunk(lax.rem(i + 1, N_DEV))
        part_ref[1] = p1
        for l in lane_order:
            p = p7 if rightward(l) else p1
            rs_buf[l, 7] = p[:, l * lw:(l + 1) * lw].astype(jnp.bfloat16)
            r = rdma(rs_buf, rs_s, rs_r, l, 7, 0, 0)
            r.start()
            sent.append(r)

        for r_off in (6, 2, 5, 3, 4, 0):
            part_ref[r_off] = pchunk(lax.rem(i + r_off, N_DEV))

        acc = [None] * LANES
        for s in range(N_DEV - 1):
            for l in lane_order:
                rdma(rs_buf, rs_s, rs_r, l, s, s, s).wait_recv()
                r_off = (6 - s) if rightward(l) else (2 + s) % N_DEV
                acc[l] = (rs_buf[l, s].astype(jnp.float32)
                          + part_ref[r_off][:, l * lw:(l + 1) * lw])
                if s < N_DEV - 2:
                    rs_buf[l, s] = acc[l].astype(jnp.bfloat16)
                    r = rdma(rs_buf, rs_s, rs_r, l, s, s + 1, s + 1)
                    r.start()
                    sent.append(r)

        for l in lane_order:
            ag_buf[l, 7] = acc[l].astype(jnp.bfloat16)
            r = rdma(ag_buf, ag_s, ag_r, l, 7, 0, 0)
            r.start()
            sent.append(r)
        own = jnp.dot(ag_buf[0, 7], w2l[0], preferred_element_type=jnp.float32)
        for l in range(1, LANES):
            own = own + jnp.dot(ag_buf[l, 7], w2l[l],
                                preferred_element_type=jnp.float32)
        out_ref[pl.ds(i * chunk, chunk), :] = own

        for s in range(N_DEV - 1):
            for l in lane_order:
                rdma(ag_buf, ag_s, ag_r, l, s, s, s).wait_recv()
                if s < N_DEV - 2:
                    r = rdma(ag_buf, ag_s, ag_r, l, s, s + 1, s + 1)
                    r.start()
                    sent.append(r)
            pieceR = jnp.dot(ag_buf[0, s], w2l[0],
                             preferred_element_type=jnp.float32)
            for l in range(1, NSUB):
                pieceR = pieceR + jnp.dot(ag_buf[l, s], w2l[l],
                                          preferred_element_type=jnp.float32)
            pieceL = jnp.dot(ag_buf[NSUB, s], w2l[NSUB],
                             preferred_element_type=jnp.float32)
            for l in range(NSUB + 1, LANES):
                pieceL = pieceL + jnp.dot(ag_buf[l, s], w2l[l],
                                          preferred_element_type=jnp.float32)
            cR = lax.rem(i - 1 - s + 2 * N_DEV, N_DEV)
            cL = lax.rem(i + 1 + s, N_DEV)
            dsR = pl.ds(cR * chunk, chunk)
            dsL = pl.ds(cL * chunk, chunk)
            if s < 3:
                out_ref[dsR, :] = pieceR
                out_ref[dsL, :] = pieceL
            elif s == 3:
                out_ref[dsR, :] = pieceR + pieceL
            else:
                out_ref[dsR, :] = out_ref[dsR, :] + pieceR
                out_ref[dsL, :] = out_ref[dsL, :] + pieceL

        for r in sent:
            r.wait_send()

    return pl.pallas_call(
        body,
        out_shape=jax.ShapeDtypeStruct((m, f), jnp.float32),
        in_specs=[
            pl.BlockSpec(memory_space=pltpu.VMEM),
            pl.BlockSpec(memory_space=pltpu.VMEM),
            pl.BlockSpec(memory_space=pltpu.VMEM),
        ],
        out_specs=pl.BlockSpec(memory_space=pltpu.VMEM),
        scratch_shapes=[
            pltpu.VMEM((N_DEV, chunk, d), jnp.float32),
            pltpu.VMEM((LANES, N_DEV, chunk, lw), jnp.bfloat16),
            pltpu.VMEM((LANES, N_DEV, chunk, lw), jnp.bfloat16),
            pltpu.SemaphoreType.DMA((LANES, N_DEV - 1)),
            pltpu.SemaphoreType.DMA((LANES, N_DEV - 1)),
            pltpu.SemaphoreType.DMA((LANES, N_DEV - 1)),
            pltpu.SemaphoreType.DMA((LANES, N_DEV - 1)),
        ],
        compiler_params=pltpu.CompilerParams(collective_id=0),
    )(x, W1, W2)
